# Initial kernel scaffold; baseline (speedup 1.0000x reference)
import jax
import jax.numpy as jnp
from jax.experimental import pallas as pl


def kernel(x, w1, b1, t1, w2, b2, t2, w3, b3, t3, w4, b4, t4, w5, b5, wfc, bfc):
    raise NotImplementedError("write your pallas kernel here")


# speedup vs baseline: 1.8500x; 1.8500x over previous
import functools

import jax
import jax.numpy as jnp
from jax.experimental import pallas as pl
from jax.experimental.pallas import tpu as pltpu


def _first_conv(x, w_ref, b_ref):
    B = x.shape[0]
    acc = jnp.zeros((B * 32 * 32, 32), jnp.float32)
    for q in range(2):
        for p in range(2):
            t = x[:, q:q + 32, p:p + 32, :].reshape(B * 32 * 32, 4)
            acc = acc + jnp.dot(t, w_ref[2 * q + p],
                                preferred_element_type=jnp.float32)
    acc = jnp.maximum(acc + b_ref[...], 0.0)
    return acc.reshape(B, 32, 32, 32).astype(jnp.bfloat16)


def _conv_layer(o, w_ref, b_ref):
    B, H, _, C = o.shape
    hb = H // 2 + 1
    OH = H // 2
    Cout = w_ref.shape[-1]
    op = jnp.pad(o, ((0, 0), (1, 1), (1, 1), (0, 0)))
    xs = op.reshape(B, hb, 2, hb, 2, C)
    acc = jnp.zeros((B * OH * OH, Cout), jnp.float32)
    for q in range(2):
        for r in range(2):
            for p in range(2):
                for c in range(2):
                    t = xs[:, q:q + OH, r, p:p + OH, c, :]
                    t = t.reshape(B * OH * OH, C)
                    acc = acc + jnp.dot(t, w_ref[(2 * q + p) * 4 + 2 * r + c],
                                        preferred_element_type=jnp.float32)
    acc = jnp.maximum(acc + b_ref[...], 0.0)
    return acc.reshape(B, OH, OH, Cout).astype(jnp.bfloat16)


def _enc_kernel(s1_ref, w1_ref, b1_ref, w2_ref, b2_ref, w3_ref, b3_ref,
                w4_ref, b4_ref, w5_ref, b5_ref, wfc_ref, bfc_ref, out_ref):
    o = _first_conv(s1_ref[...], w1_ref, b1_ref)
    o = _conv_layer(o, w2_ref, b2_ref)
    o = _conv_layer(o, w3_ref, b3_ref)
    o = _conv_layer(o, w4_ref, b4_ref)
    o = _conv_layer(o, w5_ref, b5_ref)
    B = o.shape[0]
    nout = wfc_ref.shape[-1]
    acc = jnp.broadcast_to(bfc_ref[...], (B, nout)).astype(jnp.float32)
    for i in range(2):
        for j in range(2):
            acc = acc + jnp.dot(o[:, i, j, :], wfc_ref[2 * i + j],
                                preferred_element_type=jnp.float32)
    out_ref[...] = acc


def _zero_index(ndim, b):
    return (0,) * ndim


def kernel(x, w1, b1, t1, w2, b2, t2, w3, b3, t3, w4, b4, t4, w5, b5, wfc, bfc):
    B = x.shape[0]
    out_dim = bfc.shape[-1]

    xp = jnp.pad(x[:, 0], ((0, 0), (1, 1), (1, 1)))
    s1 = xp.reshape(B, 33, 2, 33, 2).transpose(0, 1, 3, 2, 4)
    s1 = s1.reshape(B, 33, 33, 4).astype(jnp.bfloat16)

    BB = 16
    operands = (s1, w1, b1, w2, b2, w3, b3, w4, b4, w5, b5, wfc, bfc)
    in_specs = [pl.BlockSpec((BB, 33, 33, 4), lambda b: (b, 0, 0, 0))]
    for op in operands[1:]:
        in_specs.append(pl.BlockSpec(op.shape,
                                     functools.partial(_zero_index, op.ndim)))

    out = pl.pallas_call(
        _enc_kernel,
        out_shape=jax.ShapeDtypeStruct((B, out_dim), jnp.float32),
        grid=(B // BB,),
        in_specs=in_specs,
        out_specs=pl.BlockSpec((BB, out_dim), lambda b: (b, 0)),
        compiler_params=pltpu.CompilerParams(
            dimension_semantics=("parallel",),
            vmem_limit_bytes=64 * 1024 * 1024,
        ),
    )(*operands)
    return out


# speedup vs baseline: 2.7916x; 1.5089x over previous
import functools

import jax
import jax.numpy as jnp
from jax.experimental import pallas as pl
from jax.experimental.pallas import tpu as pltpu


def _first_conv(x, w_ref, b_ref):
    B = x.shape[0]
    u = jnp.concatenate(
        [x[:, q:q + 32, p:p + 32, :] for q in range(2) for p in range(2)],
        axis=-1)
    acc = jnp.dot(u.reshape(B * 32 * 32, 16), w_ref[...],
                  preferred_element_type=jnp.float32)
    acc = jnp.maximum(acc + b_ref[...], 0.0)
    return acc.reshape(B, 32, 32, 32).astype(jnp.bfloat16)


def _conv_layer(o, w_ref, b_ref):
    B, H, _, C = o.shape
    hb = H // 2 + 1
    OH = H // 2
    Cout = w_ref.shape[-1]
    op = jnp.pad(o, ((0, 0), (1, 1), (1, 1), (0, 0)))
    xs = op.reshape(B, hb, 2, hb, 2, C)
    u = jnp.concatenate(
        [xs[:, :, r, :, c, :] for r in range(2) for c in range(2)],
        axis=-1)
    acc = jnp.zeros((B * OH * OH, Cout), jnp.float32)
    for q in range(2):
        for p in range(2):
            t = u[:, q:q + OH, p:p + OH, :].reshape(B * OH * OH, 4 * C)
            acc = acc + jnp.dot(t, w_ref[2 * q + p],
                                preferred_element_type=jnp.float32)
    acc = jnp.maximum(acc + b_ref[...], 0.0)
    return acc.reshape(B, OH, OH, Cout).astype(jnp.bfloat16)


def _enc_kernel(s1_ref, w1_ref, b1_ref, w2_ref, b2_ref, w3_ref, b3_ref,
                w4_ref, b4_ref, w5_ref, b5_ref, wfc_ref, bfc_ref, out_ref):
    o = _first_conv(s1_ref[...], w1_ref, b1_ref)
    o = _conv_layer(o, w2_ref, b2_ref)
    o = _conv_layer(o, w3_ref, b3_ref)
    o = _conv_layer(o, w4_ref, b4_ref)
    o = _conv_layer(o, w5_ref, b5_ref)
    B = o.shape[0]
    feat = o.reshape(B, 512)
    acc = jnp.dot(feat, wfc_ref[...], preferred_element_type=jnp.float32)
    out_ref[...] = acc + bfc_ref[...]


def _zero_index(ndim, b):
    return (0,) * ndim


def kernel(x, w1, b1, t1, w2, b2, t2, w3, b3, t3, w4, b4, t4, w5, b5, wfc, bfc):
    B = x.shape[0]
    out_dim = bfc.shape[-1]

    xp = jnp.pad(x[:, 0], ((0, 0), (1, 1), (1, 1)))
    s1 = xp.reshape(B, 33, 2, 33, 2).transpose(0, 1, 3, 2, 4)
    s1 = s1.reshape(B, 33, 33, 4).astype(jnp.bfloat16)

    w1c = w1.reshape(16, 32)
    wc = [w.reshape(4, 4 * w.shape[1], w.shape[2]) for w in (w2, w3, w4, w5)]
    wfcc = wfc.reshape(512, out_dim)

    BB = 16
    operands = (s1, w1c, b1, wc[0], b2, wc[1], b3, wc[2], b4, wc[3], b5,
                wfcc, bfc)
    in_specs = [pl.BlockSpec((BB, 33, 33, 4), lambda b: (b, 0, 0, 0))]
    for op in operands[1:]:
        in_specs.append(pl.BlockSpec(op.shape,
                                     functools.partial(_zero_index, op.ndim)))

    out = pl.pallas_call(
        _enc_kernel,
        out_shape=jax.ShapeDtypeStruct((B, out_dim), jnp.float32),
        grid=(B // BB,),
        in_specs=in_specs,
        out_specs=pl.BlockSpec((BB, out_dim), lambda b: (b, 0)),
        compiler_params=pltpu.CompilerParams(
            dimension_semantics=("parallel",),
            vmem_limit_bytes=64 * 1024 * 1024,
        ),
    )(*operands)
    return out
